# baseline (speedup 1.0000x reference)
import functools

import jax
import jax.numpy as jnp
from jax import lax
from jax.experimental import pallas as pl
from jax.experimental.pallas import tpu as pltpu
from jax.experimental.pallas import tpu_sc as plsc

P = 2
C = 768
HW = 576
ROWS = 16 * C
NC, NS, L = 2, 16, 16
NW = NC * NS
RPW = ROWS // NW
CHUNK = 96
NCHUNK = RPW // CHUNK


def _body(x_hbm, w_hbm, thr_hbm, out_hbm, wbuf, thrbuf, idxbuf, rows, sem):
    cid = lax.axis_index("c")
    sid = lax.axis_index("s")
    wid = sid * NC + cid
    s = wid >> 1
    half = wid & 1
    c0 = half * RPW
    p = s >> 3
    self_base = s * C
    other_base = (s ^ 8) * C

    pltpu.sync_copy(w_hbm.at[pl.ds(p * C + c0, RPW)], wbuf)
    pltpu.sync_copy(thr_hbm, thrbuf)
    thr = thrbuf[...]

    for k in range(NCHUNK):
        for j in range(CHUNK // L):
            off = k * CHUNK + j * L
            w = wbuf[pl.ds(off, L)]
            c = c0 + off + lax.iota(jnp.int32, L)
            m = jnp.abs(w) < thr
            idxbuf[pl.ds(j * L, L)] = jnp.where(m, other_base + c, self_base + c)
        pltpu.async_copy(x_hbm.at[idxbuf], rows, sem).wait()
        pltpu.sync_copy(rows, out_hbm.at[pl.ds(wid * RPW + k * CHUNK, CHUNK)])


@functools.partial(jax.jit, static_argnums=())
def _exchange(xf, wf, thr16):
    mesh = plsc.VectorSubcoreMesh(
        core_axis_name="c", subcore_axis_name="s", num_cores=NC, num_subcores=NS
    )
    return pl.kernel(
        _body,
        out_type=jax.ShapeDtypeStruct((ROWS, HW), jnp.float32),
        mesh=mesh,
        scratch_types=[
            pltpu.VMEM((RPW,), jnp.float32),
            pltpu.VMEM((L,), jnp.float32),
            pltpu.VMEM((CHUNK,), jnp.int32),
            pltpu.VMEM((CHUNK, HW), jnp.float32),
            pltpu.SemaphoreType.DMA,
        ],
        compiler_params=pltpu.CompilerParams(use_tc_tiling_on_sc=False),
    )(xf, wf, thr16)


def kernel(x, bn_weight, bn_threshold):
    xf = x.reshape(ROWS, HW)
    wf = bn_weight.reshape(P * C)
    thr16 = jnp.full((L,), bn_threshold, dtype=jnp.float32)
    out = _exchange(xf, wf, thr16)
    return out.reshape(x.shape)


# speedup vs baseline: 8.2690x; 8.2690x over previous
import functools

import jax
import jax.numpy as jnp
from jax.experimental import pallas as pl
from jax.experimental.pallas import tpu as pltpu

S = 16
C = 768
HW = 576
BR = 144
NB = HW // BR


def _body(thr_ref, w_ref, xs_ref, xo_ref, o_ref):
    thr = thr_ref[0]
    m0 = (jnp.abs(w_ref[0:1, :]) < thr)[:, None, :]
    m1 = (jnp.abs(w_ref[1:2, :]) < thr)[:, None, :]
    xs = xs_ref[...]
    xo = xo_ref[...]
    o_ref[0] = jnp.where(m0, xo, xs)
    o_ref[1] = jnp.where(m1, xs, xo)


@jax.jit
def _exchange(xt, w, thr):
    return pl.pallas_call(
        _body,
        grid=(8, NB),
        in_specs=[
            pl.BlockSpec(memory_space=pltpu.SMEM),
            pl.BlockSpec((2, C), lambda s, i: (0, 0)),
            pl.BlockSpec((1, BR, C), lambda s, i: (s, i, 0)),
            pl.BlockSpec((1, BR, C), lambda s, i: (s + 8, i, 0)),
        ],
        out_specs=pl.BlockSpec((2, 1, BR, C), lambda s, i: (0, s, i, 0)),
        out_shape=jax.ShapeDtypeStruct((2, 8, HW, C), jnp.float32),
        compiler_params=pltpu.CompilerParams(
            dimension_semantics=("parallel", "parallel"),
        ),
    )(thr, w, xt, xt)


def kernel(x, bn_weight, bn_threshold):
    xt = x.transpose(0, 2, 3, 1).reshape(S, HW, C)
    thr = jnp.full((1,), bn_threshold, dtype=jnp.float32)
    out = _exchange(xt, bn_weight, thr)
    return out.reshape(S, 24, 24, C).transpose(0, 3, 1, 2)


# speedup vs baseline: 12.8536x; 1.5544x over previous
import functools

import jax
import jax.numpy as jnp
from jax.experimental import pallas as pl
from jax.experimental.pallas import tpu as pltpu

S = 16
C = 768
HW = 576
BR = 576
NB = HW // BR


def _body(thr_ref, w_ref, xs_ref, xo_ref, o_ref):
    thr = thr_ref[0]
    m0 = (jnp.abs(w_ref[0:1, :]) < thr)[:, None, :]
    m1 = (jnp.abs(w_ref[1:2, :]) < thr)[:, None, :]
    xs = xs_ref[...]
    xo = xo_ref[...]
    o_ref[0] = jnp.where(m0, xo, xs)
    o_ref[1] = jnp.where(m1, xs, xo)


@jax.jit
def _exchange(xt, w, thr):
    return pl.pallas_call(
        _body,
        grid=(8, NB),
        in_specs=[
            pl.BlockSpec(memory_space=pltpu.SMEM),
            pl.BlockSpec((2, C), lambda s, i: (0, 0)),
            pl.BlockSpec((1, BR, C), lambda s, i: (s, i, 0)),
            pl.BlockSpec((1, BR, C), lambda s, i: (s + 8, i, 0)),
        ],
        out_specs=pl.BlockSpec((2, 1, BR, C), lambda s, i: (0, s, i, 0)),
        out_shape=jax.ShapeDtypeStruct((2, 8, HW, C), jnp.float32),
        compiler_params=pltpu.CompilerParams(
            dimension_semantics=("parallel", "parallel"),
        ),
    )(thr, w, xt, xt)


def kernel(x, bn_weight, bn_threshold):
    xt = x.transpose(0, 2, 3, 1).reshape(S, HW, C)
    thr = jnp.full((1,), bn_threshold, dtype=jnp.float32)
    out = _exchange(xt, bn_weight, thr)
    return out.reshape(S, 24, 24, C).transpose(0, 3, 1, 2)
